# baseline (speedup 1.0000x reference)
import functools

import jax
import jax.numpy as jnp
from jax import lax
from jax.experimental import pallas as pl
from jax.experimental.pallas import tpu as pltpu
from jax.experimental.pallas import tpu_sc as plsc

N = 10000
NPAD = 10240
E = 160000
EPAD = 163840
KP1 = 384
DHID = 512
DP2 = 384
CW = 128
NCH1 = DHID // CW
NCH2 = DP2 // CW
NC, NS = 2, 16
RPT = NPAD // NS
RBLK = 256
GRID = NPAD // RBLK

_MESH = dict(core_axis_name="c", subcore_axis_name="s", num_cores=NC,
             num_subcores=NS)
_F32 = jnp.float32


@functools.partial(
    pl.kernel,
    out_type=[jax.ShapeDtypeStruct((NPAD, CW), _F32) for _ in range(NC)],
    mesh=plsc.VectorSubcoreMesh(**_MESH),
    scratch_types=[
        pltpu.VMEM((40, 128), jnp.int32),
        pltpu.VMEM((128, CW), _F32),
        pltpu.VMEM((16, CW), _F32),
        pltpu.VMEM_SHARED((NPAD, CW), _F32),
    ],
)
def _deg_kernel(dst_hbm, out0, out1, idx_v, ones_v, zero_v, acc):
    cid = lax.axis_index("c")
    sid = lax.axis_index("s")
    wid = sid * NC + cid

    def fill(i, _):
        for k16 in range(CW // 16):
            ones_v[i, pl.ds(k16 * 16, 16)] = jnp.ones((16,), _F32)
        return 0
    lax.fori_loop(0, 128, fill, 0)

    def fillz(i, _):
        for k16 in range(CW // 16):
            zero_v[i, pl.ds(k16 * 16, 16)] = jnp.zeros((16,), _F32)
        return 0
    lax.fori_loop(0, 16, fillz, 0)

    pltpu.sync_copy(dst_hbm.at[wid], idx_v)

    def zero(i, _):
        pltpu.sync_copy(zero_v, acc.at[pl.ds(sid * RPT + i * 16, 16)])
        return 0
    lax.fori_loop(0, RPT // 16, zero, 0)
    plsc.subcore_barrier()

    def scat(g, _):
        pltpu.sync_copy(ones_v, acc.at[idx_v.at[g]], add=True)
        return 0
    lax.fori_loop(0, 40, scat, 0)
    plsc.subcore_barrier()

    @pl.when(cid == 0)
    def _():
        pltpu.sync_copy(acc.at[pl.ds(sid * RPT, RPT)],
                        out0.at[pl.ds(sid * RPT, RPT)])

    @pl.when(cid == 1)
    def _():
        pltpu.sync_copy(acc.at[pl.ds(sid * RPT, RPT)],
                        out1.at[pl.ds(sid * RPT, RPT)])


def _make_agg(nchunks, split_last):
    nouts = nchunks + (1 if split_last else 0)

    @functools.partial(
        pl.kernel,
        out_type=[jax.ShapeDtypeStruct((NPAD, CW), _F32)
                  for _ in range(nouts)],
        mesh=plsc.VectorSubcoreMesh(**_MESH),
        scratch_types=[
            pltpu.VMEM((40, 128), jnp.int32),
            pltpu.VMEM((40, 128), jnp.int32),
            pltpu.VMEM((128, CW), _F32),
            pltpu.VMEM((128, CW), _F32),
            pltpu.VMEM((16, CW), _F32),
            pltpu.VMEM_SHARED((NPAD, CW), _F32),
            pltpu.SemaphoreType.DMA,
            pltpu.SemaphoreType.DMA,
            pltpu.SemaphoreType.DMA,
            pltpu.SemaphoreType.DMA,
        ],
    )
    def agg(*refs):
        src_hbm, dst_hbm = refs[0], refs[1]
        tables = refs[2:2 + nchunks]
        outs = refs[2 + nchunks:2 + nchunks + nouts]
        (src_v, dst_v, rows_a, rows_b, zero_v, acc,
         sem_a, sem_b, sem_sa, sem_sb) = refs[2 + nchunks + nouts:]
        cid = lax.axis_index("c")
        sid = lax.axis_index("s")

        def fz(i, _):
            for k16 in range(CW // 16):
                zero_v[i, pl.ds(k16 * 16, 16)] = jnp.zeros((16,), _F32)
            return 0
        lax.fori_loop(0, 16, fz, 0)

        def run_chunk(table, out, halves):
            def zero(i, _):
                pltpu.sync_copy(zero_v,
                                acc.at[pl.ds(sid * RPT + i * 16, 16)])
                return 0
            lax.fori_loop(0, RPT // 16, zero, 0)
            plsc.subcore_barrier()

            for half in halves:
                pltpu.sync_copy(src_hbm.at[sid, pl.ds(half * 40, 40)],
                                src_v)
                pltpu.sync_copy(dst_hbm.at[sid, pl.ds(half * 40, 40)],
                                dst_v)

                def g_start(buf, sem, g):
                    pltpu.async_copy(table.at[src_v.at[g]], buf, sem)

                def g_wait(buf, sem):
                    pltpu.make_async_copy(table.at[src_v.at[0]], buf,
                                          sem).wait()

                def s_start(buf, sem, g):
                    pltpu.async_copy(buf, acc.at[dst_v.at[g]], sem,
                                     add=True)

                def s_wait(buf, sem):
                    pltpu.make_async_copy(buf, acc.at[dst_v.at[0]],
                                          sem).wait()

                g_start(rows_a, sem_a, 0)
                g_start(rows_b, sem_b, 1)

                def body(gg, _):
                    g0 = 2 * gg
                    g_wait(rows_a, sem_a)
                    s_start(rows_a, sem_sa, g0)
                    g_wait(rows_b, sem_b)
                    s_start(rows_b, sem_sb, g0 + 1)
                    s_wait(rows_a, sem_sa)

                    @pl.when(gg < 19)
                    def _():
                        g_start(rows_a, sem_a, g0 + 2)
                    s_wait(rows_b, sem_sb)

                    @pl.when(gg < 19)
                    def _():
                        g_start(rows_b, sem_b, g0 + 3)
                    return 0
                lax.fori_loop(0, 20, body, 0)

            plsc.subcore_barrier()
            pltpu.sync_copy(acc.at[pl.ds(sid * RPT, RPT)],
                            out.at[pl.ds(sid * RPT, RPT)])
            plsc.subcore_barrier()

        half_chunks = nchunks // 2
        for core_val in range(NC):
            @pl.when(cid == core_val)
            def _():
                for j in range(half_chunks):
                    chunk = core_val * half_chunks + j
                    run_chunk(tables[chunk], outs[chunk], (0, 1))
                if split_last:
                    run_chunk(tables[nchunks - 1],
                              outs[nchunks - 1 + core_val], (core_val,))

    return agg


_agg1 = _make_agg(NCH1, False)
_agg2 = _make_agg(NCH2, True)


def _dis_of(c0_ref, c1_ref):
    cnt = c0_ref[:, :1] + c1_ref[:, :1]
    return lax.rsqrt(cnt + 1.0)


def _b1_body(*refs):
    x_ref, w_ref, c0_ref, c1_ref = refs[:4]
    outs = refs[4:]
    dis = _dis_of(c0_ref, c1_ref)
    h = lax.dot_general(x_ref[...], w_ref[...], (((0,), (0,)), ((), ())),
                        preferred_element_type=_F32)
    hp = h * dis
    for c, o in enumerate(outs):
        o[...] = hp[:, c * CW:(c + 1) * CW]


_b1_call = pl.pallas_call(
    _b1_body,
    grid=(GRID,),
    in_specs=[
        pl.BlockSpec((KP1, RBLK), lambda i: (0, i)),
        pl.BlockSpec((KP1, DHID), lambda i: (0, 0)),
        pl.BlockSpec((RBLK, CW), lambda i: (i, 0)),
        pl.BlockSpec((RBLK, CW), lambda i: (i, 0)),
    ],
    out_specs=[pl.BlockSpec((RBLK, CW), lambda i: (i, 0))
               for _ in range(NCH1)],
    out_shape=[jax.ShapeDtypeStruct((NPAD, CW), _F32) for _ in range(NCH1)],
)


def _b2_body(*refs):
    aggs = refs[:NCH1]
    hps = refs[NCH1:2 * NCH1]
    c0_ref, c1_ref, b1_ref, w_ref = refs[2 * NCH1:2 * NCH1 + 4]
    outs = refs[2 * NCH1 + 4:]
    dis = _dis_of(c0_ref, c1_ref)
    zs = []
    for c, (a, h) in enumerate(zip(aggs, hps)):
        pre = dis * (a[...] + h[...]) + b1_ref[0:1, c * CW:(c + 1) * CW]
        zs.append(jnp.maximum(pre, 0.0))
    z = jnp.concatenate(zs, axis=1)
    h2f = jnp.dot(z, w_ref[...], preferred_element_type=_F32)
    for c, o in enumerate(outs):
        o[...] = dis * h2f[:, c * CW:(c + 1) * CW]


_b2_call = pl.pallas_call(
    _b2_body,
    grid=(GRID,),
    in_specs=(
        [pl.BlockSpec((RBLK, CW), lambda i: (i, 0)) for _ in range(2 * NCH1)]
        + [pl.BlockSpec((RBLK, CW), lambda i: (i, 0)) for _ in range(2)]
        + [pl.BlockSpec((1, DHID), lambda i: (0, 0)),
           pl.BlockSpec((DHID, DP2), lambda i: (0, 0))]
    ),
    out_specs=[pl.BlockSpec((RBLK, CW), lambda i: (i, 0))
               for _ in range(NCH2)],
    out_shape=[jax.ShapeDtypeStruct((NPAD, CW), _F32) for _ in range(NCH2)],
)


def _b3_body(*refs):
    aggs = refs[:NCH2 + 1]
    hps = refs[NCH2 + 1:2 * NCH2 + 1]
    c0_ref, c1_ref, b2_ref = refs[2 * NCH2 + 1:2 * NCH2 + 4]
    o = refs[-1]
    dis = _dis_of(c0_ref, c1_ref)
    svals = [aggs[0][...], aggs[1][...], aggs[2][...] + aggs[3][...]]
    outs = []
    for c, (a, h) in enumerate(zip(svals, hps)):
        pre = dis * (a + h[...]) + b2_ref[0:1, c * CW:(c + 1) * CW]
        outs.append(jnp.maximum(pre, 0.0))
    o[...] = jnp.transpose(jnp.concatenate(outs, axis=1))


_b3_call = pl.pallas_call(
    _b3_body,
    grid=(GRID,),
    in_specs=(
        [pl.BlockSpec((RBLK, CW), lambda i: (i, 0)) for _ in range(2 * NCH2 + 1)]
        + [pl.BlockSpec((RBLK, CW), lambda i: (i, 0)) for _ in range(2)]
        + [pl.BlockSpec((1, DP2), lambda i: (0, 0))]
    ),
    out_specs=pl.BlockSpec((DP2, RBLK), lambda i: (0, i)),
    out_shape=jax.ShapeDtypeStruct((DP2, NPAD), _F32),
)


def kernel(x, edge_index, W1, b1, W2, b2):
    src = edge_index[0].astype(jnp.int32)
    dst = edge_index[1].astype(jnp.int32)
    pad = jnp.full((EPAD - E,), N, jnp.int32)
    srcp = jnp.concatenate([src, pad])
    dstp = jnp.concatenate([dst, pad])
    src16 = srcp.reshape(NS, 80, 128)
    dst16 = dstp.reshape(NS, 80, 128)
    dst32 = dstp.reshape(NC * NS, 40, 128)

    xtp = jnp.pad(x.astype(_F32).T, ((0, KP1 - x.shape[1]), (0, NPAD - N)))
    w1p = jnp.pad(W1.astype(_F32), ((0, KP1 - W1.shape[0]), (0, 0)))
    w2p = jnp.pad(W2.astype(_F32), ((0, 0), (0, DP2 - W2.shape[1])))
    b1r = b1.reshape(1, DHID).astype(_F32)
    b2r = jnp.pad(b2.astype(_F32), (0, DP2 - b2.shape[0])).reshape(1, DP2)

    c0, c1 = _deg_kernel(dst32)
    h1 = _b1_call(xtp, w1p, c0, c1)
    a1 = _agg1(src16, dst16, *h1)
    h2 = _b2_call(*a1, *h1, c0, c1, b1r, w2p)
    a2 = _agg2(src16, dst16, *h2)
    out = _b3_call(*a2, *h2, c0, c1, b2r)
    return out[:300, :N].T


# speedup vs baseline: 1.0237x; 1.0237x over previous
import functools

import jax
import jax.numpy as jnp
from jax import lax
from jax.experimental import pallas as pl
from jax.experimental.pallas import tpu as pltpu
from jax.experimental.pallas import tpu_sc as plsc

N = 10000
NPAD = 10240
E = 160000
EPAD = 163840
KP1 = 384
DHID = 512
DP2 = 384
CW = 128
NCH1 = DHID // CW
NCH2 = DP2 // CW
NC, NS = 2, 16
RPT = NPAD // NS
RBLK = 512
GRID = NPAD // RBLK

_MESH = dict(core_axis_name="c", subcore_axis_name="s", num_cores=NC,
             num_subcores=NS)
_F32 = jnp.float32


@functools.partial(
    pl.kernel,
    out_type=[jax.ShapeDtypeStruct((NPAD, CW), _F32) for _ in range(NC)],
    mesh=plsc.VectorSubcoreMesh(**_MESH),
    scratch_types=[
        pltpu.VMEM((40, 128), jnp.int32),
        pltpu.VMEM((128, CW), _F32),
        pltpu.VMEM((16, CW), _F32),
        pltpu.VMEM_SHARED((NPAD, CW), _F32),
    ],
)
def _deg_kernel(dst_hbm, out0, out1, idx_v, ones_v, zero_v, acc):
    cid = lax.axis_index("c")
    sid = lax.axis_index("s")
    wid = sid * NC + cid

    def fill(i, _):
        for k16 in range(CW // 16):
            ones_v[i, pl.ds(k16 * 16, 16)] = jnp.ones((16,), _F32)
        return 0
    lax.fori_loop(0, 128, fill, 0)

    def fillz(i, _):
        for k16 in range(CW // 16):
            zero_v[i, pl.ds(k16 * 16, 16)] = jnp.zeros((16,), _F32)
        return 0
    lax.fori_loop(0, 16, fillz, 0)

    pltpu.sync_copy(dst_hbm.at[wid], idx_v)

    def zero(i, _):
        pltpu.sync_copy(zero_v, acc.at[pl.ds(sid * RPT + i * 16, 16)])
        return 0
    lax.fori_loop(0, RPT // 16, zero, 0)
    plsc.subcore_barrier()

    def scat(g, _):
        pltpu.sync_copy(ones_v, acc.at[idx_v.at[g]], add=True)
        return 0
    lax.fori_loop(0, 40, scat, 0)
    plsc.subcore_barrier()

    @pl.when(cid == 0)
    def _():
        pltpu.sync_copy(acc.at[pl.ds(sid * RPT, RPT)],
                        out0.at[pl.ds(sid * RPT, RPT)])

    @pl.when(cid == 1)
    def _():
        pltpu.sync_copy(acc.at[pl.ds(sid * RPT, RPT)],
                        out1.at[pl.ds(sid * RPT, RPT)])


def _make_agg(nchunks, split_last):
    nouts = nchunks + (1 if split_last else 0)

    @functools.partial(
        pl.kernel,
        out_type=[jax.ShapeDtypeStruct((NPAD, CW), _F32)
                  for _ in range(nouts)],
        mesh=plsc.VectorSubcoreMesh(**_MESH),
        scratch_types=[
            pltpu.VMEM((40, 128), jnp.int32),
            pltpu.VMEM((40, 128), jnp.int32),
            pltpu.VMEM((128, CW), _F32),
            pltpu.VMEM((128, CW), _F32),
            pltpu.VMEM((16, CW), _F32),
            pltpu.VMEM_SHARED((NPAD, CW), _F32),
            pltpu.SemaphoreType.DMA,
            pltpu.SemaphoreType.DMA,
            pltpu.SemaphoreType.DMA,
            pltpu.SemaphoreType.DMA,
        ],
    )
    def agg(*refs):
        src_hbm, dst_hbm = refs[0], refs[1]
        tables = refs[2:2 + nchunks]
        outs = refs[2 + nchunks:2 + nchunks + nouts]
        (src_v, dst_v, rows_a, rows_b, zero_v, acc,
         sem_a, sem_b, sem_sa, sem_sb) = refs[2 + nchunks + nouts:]
        cid = lax.axis_index("c")
        sid = lax.axis_index("s")

        def fz(i, _):
            for k16 in range(CW // 16):
                zero_v[i, pl.ds(k16 * 16, 16)] = jnp.zeros((16,), _F32)
            return 0
        lax.fori_loop(0, 16, fz, 0)

        def run_chunk(table, out, halves):
            def zero(i, _):
                pltpu.sync_copy(zero_v,
                                acc.at[pl.ds(sid * RPT + i * 16, 16)])
                return 0
            lax.fori_loop(0, RPT // 16, zero, 0)
            plsc.subcore_barrier()

            for half in halves:
                pltpu.sync_copy(src_hbm.at[sid, pl.ds(half * 40, 40)],
                                src_v)
                pltpu.sync_copy(dst_hbm.at[sid, pl.ds(half * 40, 40)],
                                dst_v)

                def g_start(buf, sem, g):
                    pltpu.async_copy(table.at[src_v.at[g]], buf, sem)

                def g_wait(buf, sem):
                    pltpu.make_async_copy(table.at[src_v.at[0]], buf,
                                          sem).wait()

                def s_start(buf, sem, g):
                    pltpu.async_copy(buf, acc.at[dst_v.at[g]], sem,
                                     add=True)

                def s_wait(buf, sem):
                    pltpu.make_async_copy(buf, acc.at[dst_v.at[0]],
                                          sem).wait()

                g_start(rows_a, sem_a, 0)
                g_start(rows_b, sem_b, 1)

                def body(gg, _):
                    g0 = 2 * gg
                    g_wait(rows_a, sem_a)
                    s_start(rows_a, sem_sa, g0)
                    g_wait(rows_b, sem_b)
                    s_start(rows_b, sem_sb, g0 + 1)
                    s_wait(rows_a, sem_sa)

                    @pl.when(gg < 19)
                    def _():
                        g_start(rows_a, sem_a, g0 + 2)
                    s_wait(rows_b, sem_sb)

                    @pl.when(gg < 19)
                    def _():
                        g_start(rows_b, sem_b, g0 + 3)
                    return 0
                lax.fori_loop(0, 20, body, 0)

            plsc.subcore_barrier()
            pltpu.sync_copy(acc.at[pl.ds(sid * RPT, RPT)],
                            out.at[pl.ds(sid * RPT, RPT)])
            plsc.subcore_barrier()

        half_chunks = nchunks // 2
        for core_val in range(NC):
            @pl.when(cid == core_val)
            def _():
                for j in range(half_chunks):
                    chunk = core_val * half_chunks + j
                    run_chunk(tables[chunk], outs[chunk], (0, 1))
                if split_last:
                    run_chunk(tables[nchunks - 1],
                              outs[nchunks - 1 + core_val], (core_val,))

    return agg


_agg1 = _make_agg(NCH1, False)
_agg2 = _make_agg(NCH2, True)


def _dis_of(c0_ref, c1_ref):
    cnt = c0_ref[:, :1] + c1_ref[:, :1]
    return lax.rsqrt(cnt + 1.0)


def _b1_body(*refs):
    x_ref, w_ref, c0_ref, c1_ref = refs[:4]
    outs = refs[4:]
    dis = _dis_of(c0_ref, c1_ref)
    h = lax.dot_general(x_ref[...], w_ref[...], (((0,), (0,)), ((), ())),
                        preferred_element_type=_F32)
    hp = h * dis
    for c, o in enumerate(outs):
        o[...] = hp[:, c * CW:(c + 1) * CW]


_b1_call = pl.pallas_call(
    _b1_body,
    grid=(GRID,),
    in_specs=[
        pl.BlockSpec((KP1, RBLK), lambda i: (0, i)),
        pl.BlockSpec((KP1, DHID), lambda i: (0, 0)),
        pl.BlockSpec((RBLK, CW), lambda i: (i, 0)),
        pl.BlockSpec((RBLK, CW), lambda i: (i, 0)),
    ],
    out_specs=[pl.BlockSpec((RBLK, CW), lambda i: (i, 0))
               for _ in range(NCH1)],
    out_shape=[jax.ShapeDtypeStruct((NPAD, CW), _F32) for _ in range(NCH1)],
)


def _b2_body(*refs):
    aggs = refs[:NCH1]
    hps = refs[NCH1:2 * NCH1]
    c0_ref, c1_ref, b1_ref, w_ref = refs[2 * NCH1:2 * NCH1 + 4]
    outs = refs[2 * NCH1 + 4:]
    dis = _dis_of(c0_ref, c1_ref)
    zs = []
    for c, (a, h) in enumerate(zip(aggs, hps)):
        pre = dis * (a[...] + h[...]) + b1_ref[0:1, c * CW:(c + 1) * CW]
        zs.append(jnp.maximum(pre, 0.0))
    z = jnp.concatenate(zs, axis=1)
    h2f = jnp.dot(z, w_ref[...], preferred_element_type=_F32)
    for c, o in enumerate(outs):
        o[...] = dis * h2f[:, c * CW:(c + 1) * CW]


_b2_call = pl.pallas_call(
    _b2_body,
    grid=(GRID,),
    in_specs=(
        [pl.BlockSpec((RBLK, CW), lambda i: (i, 0)) for _ in range(2 * NCH1)]
        + [pl.BlockSpec((RBLK, CW), lambda i: (i, 0)) for _ in range(2)]
        + [pl.BlockSpec((1, DHID), lambda i: (0, 0)),
           pl.BlockSpec((DHID, DP2), lambda i: (0, 0))]
    ),
    out_specs=[pl.BlockSpec((RBLK, CW), lambda i: (i, 0))
               for _ in range(NCH2)],
    out_shape=[jax.ShapeDtypeStruct((NPAD, CW), _F32) for _ in range(NCH2)],
)


def _b3_body(*refs):
    aggs = refs[:NCH2 + 1]
    hps = refs[NCH2 + 1:2 * NCH2 + 1]
    c0_ref, c1_ref, b2_ref = refs[2 * NCH2 + 1:2 * NCH2 + 4]
    o = refs[-1]
    dis = _dis_of(c0_ref, c1_ref)
    svals = [aggs[0][...], aggs[1][...], aggs[2][...] + aggs[3][...]]
    outs = []
    for c, (a, h) in enumerate(zip(svals, hps)):
        pre = dis * (a + h[...]) + b2_ref[0:1, c * CW:(c + 1) * CW]
        outs.append(jnp.maximum(pre, 0.0))
    o[...] = jnp.transpose(jnp.concatenate(outs, axis=1))


_b3_call = pl.pallas_call(
    _b3_body,
    grid=(GRID,),
    in_specs=(
        [pl.BlockSpec((RBLK, CW), lambda i: (i, 0)) for _ in range(2 * NCH2 + 1)]
        + [pl.BlockSpec((RBLK, CW), lambda i: (i, 0)) for _ in range(2)]
        + [pl.BlockSpec((1, DP2), lambda i: (0, 0))]
    ),
    out_specs=pl.BlockSpec((DP2, RBLK), lambda i: (0, i)),
    out_shape=jax.ShapeDtypeStruct((DP2, NPAD), _F32),
)


def kernel(x, edge_index, W1, b1, W2, b2):
    src = edge_index[0].astype(jnp.int32)
    dst = edge_index[1].astype(jnp.int32)
    pad = jnp.full((EPAD - E,), N, jnp.int32)
    srcp = jnp.concatenate([src, pad])
    dstp = jnp.concatenate([dst, pad])
    src16 = srcp.reshape(NS, 80, 128)
    dst16 = dstp.reshape(NS, 80, 128)
    dst32 = dstp.reshape(NC * NS, 40, 128)

    xtp = jnp.pad(x.astype(_F32).T, ((0, KP1 - x.shape[1]), (0, NPAD - N)))
    w1p = jnp.pad(W1.astype(_F32), ((0, KP1 - W1.shape[0]), (0, 0)))
    w2p = jnp.pad(W2.astype(_F32), ((0, 0), (0, DP2 - W2.shape[1])))
    b1r = b1.reshape(1, DHID).astype(_F32)
    b2r = jnp.pad(b2.astype(_F32), (0, DP2 - b2.shape[0])).reshape(1, DP2)

    c0, c1 = _deg_kernel(dst32)
    h1 = _b1_call(xtp, w1p, c0, c1)
    a1 = _agg1(src16, dst16, *h1)
    h2 = _b2_call(*a1, *h1, c0, c1, b1r, w2p)
    a2 = _agg2(src16, dst16, *h2)
    out = _b3_call(*a2, *h2, c0, c1, b2r)
    return out[:300, :N].T


# speedup vs baseline: 1.0359x; 1.0120x over previous
import functools

import jax
import jax.numpy as jnp
from jax import lax
from jax.experimental import pallas as pl
from jax.experimental.pallas import tpu as pltpu
from jax.experimental.pallas import tpu_sc as plsc

N = 10000
NPAD = 10240
E = 160000
EPAD = 163840
KP1 = 384
DHID = 512
DP2 = 384
CW = 128
NCH1 = DHID // CW
NCH2 = DP2 // CW
NC, NS = 2, 16
RPT = NPAD // NS
RBLK = 1024
GRID = NPAD // RBLK

_MESH = dict(core_axis_name="c", subcore_axis_name="s", num_cores=NC,
             num_subcores=NS)
_F32 = jnp.float32


@functools.partial(
    pl.kernel,
    out_type=[jax.ShapeDtypeStruct((NPAD, CW), _F32) for _ in range(NC)],
    mesh=plsc.VectorSubcoreMesh(**_MESH),
    scratch_types=[
        pltpu.VMEM((40, 128), jnp.int32),
        pltpu.VMEM((128, CW), _F32),
        pltpu.VMEM((16, CW), _F32),
        pltpu.VMEM_SHARED((NPAD, CW), _F32),
    ],
)
def _deg_kernel(dst_hbm, out0, out1, idx_v, ones_v, zero_v, acc):
    cid = lax.axis_index("c")
    sid = lax.axis_index("s")
    wid = sid * NC + cid

    def fill(i, _):
        for k16 in range(CW // 16):
            ones_v[i, pl.ds(k16 * 16, 16)] = jnp.ones((16,), _F32)
        return 0
    lax.fori_loop(0, 128, fill, 0)

    def fillz(i, _):
        for k16 in range(CW // 16):
            zero_v[i, pl.ds(k16 * 16, 16)] = jnp.zeros((16,), _F32)
        return 0
    lax.fori_loop(0, 16, fillz, 0)

    pltpu.sync_copy(dst_hbm.at[wid], idx_v)

    def zero(i, _):
        pltpu.sync_copy(zero_v, acc.at[pl.ds(sid * RPT + i * 16, 16)])
        return 0
    lax.fori_loop(0, RPT // 16, zero, 0)
    plsc.subcore_barrier()

    def scat(g, _):
        pltpu.sync_copy(ones_v, acc.at[idx_v.at[g]], add=True)
        return 0
    lax.fori_loop(0, 40, scat, 0)
    plsc.subcore_barrier()

    @pl.when(cid == 0)
    def _():
        pltpu.sync_copy(acc.at[pl.ds(sid * RPT, RPT)],
                        out0.at[pl.ds(sid * RPT, RPT)])

    @pl.when(cid == 1)
    def _():
        pltpu.sync_copy(acc.at[pl.ds(sid * RPT, RPT)],
                        out1.at[pl.ds(sid * RPT, RPT)])


def _make_agg(nchunks, split_last):
    nouts = nchunks + (1 if split_last else 0)

    @functools.partial(
        pl.kernel,
        out_type=[jax.ShapeDtypeStruct((NPAD, CW), _F32)
                  for _ in range(nouts)],
        mesh=plsc.VectorSubcoreMesh(**_MESH),
        scratch_types=[
            pltpu.VMEM((40, 128), jnp.int32),
            pltpu.VMEM((40, 128), jnp.int32),
            pltpu.VMEM((128, CW), _F32),
            pltpu.VMEM((128, CW), _F32),
            pltpu.VMEM((16, CW), _F32),
            pltpu.VMEM_SHARED((NPAD, CW), _F32),
            pltpu.SemaphoreType.DMA,
            pltpu.SemaphoreType.DMA,
            pltpu.SemaphoreType.DMA,
            pltpu.SemaphoreType.DMA,
        ],
    )
    def agg(*refs):
        src_hbm, dst_hbm = refs[0], refs[1]
        tables = refs[2:2 + nchunks]
        outs = refs[2 + nchunks:2 + nchunks + nouts]
        (src_v, dst_v, rows_a, rows_b, zero_v, acc,
         sem_a, sem_b, sem_sa, sem_sb) = refs[2 + nchunks + nouts:]
        cid = lax.axis_index("c")
        sid = lax.axis_index("s")

        def fz(i, _):
            for k16 in range(CW // 16):
                zero_v[i, pl.ds(k16 * 16, 16)] = jnp.zeros((16,), _F32)
            return 0
        lax.fori_loop(0, 16, fz, 0)

        def run_chunk(table, out, halves):
            def zero(i, _):
                pltpu.sync_copy(zero_v,
                                acc.at[pl.ds(sid * RPT + i * 16, 16)])
                return 0
            lax.fori_loop(0, RPT // 16, zero, 0)
            plsc.subcore_barrier()

            for half in halves:
                pltpu.sync_copy(src_hbm.at[sid, pl.ds(half * 40, 40)],
                                src_v)
                pltpu.sync_copy(dst_hbm.at[sid, pl.ds(half * 40, 40)],
                                dst_v)

                def g_start(buf, sem, g):
                    pltpu.async_copy(table.at[src_v.at[g]], buf, sem)

                def g_wait(buf, sem):
                    pltpu.make_async_copy(table.at[src_v.at[0]], buf,
                                          sem).wait()

                def s_start(buf, sem, g):
                    pltpu.async_copy(buf, acc.at[dst_v.at[g]], sem,
                                     add=True)

                def s_wait(buf, sem):
                    pltpu.make_async_copy(buf, acc.at[dst_v.at[0]],
                                          sem).wait()

                g_start(rows_a, sem_a, 0)
                g_start(rows_b, sem_b, 1)

                def body(gg, _):
                    g0 = 2 * gg
                    g_wait(rows_a, sem_a)
                    s_start(rows_a, sem_sa, g0)
                    g_wait(rows_b, sem_b)
                    s_start(rows_b, sem_sb, g0 + 1)
                    s_wait(rows_a, sem_sa)

                    @pl.when(gg < 19)
                    def _():
                        g_start(rows_a, sem_a, g0 + 2)
                    s_wait(rows_b, sem_sb)

                    @pl.when(gg < 19)
                    def _():
                        g_start(rows_b, sem_b, g0 + 3)
                    return 0
                lax.fori_loop(0, 20, body, 0)

            plsc.subcore_barrier()
            pltpu.sync_copy(acc.at[pl.ds(sid * RPT, RPT)],
                            out.at[pl.ds(sid * RPT, RPT)])
            plsc.subcore_barrier()

        half_chunks = nchunks // 2
        for core_val in range(NC):
            @pl.when(cid == core_val)
            def _():
                for j in range(half_chunks):
                    chunk = core_val * half_chunks + j
                    run_chunk(tables[chunk], outs[chunk], (0, 1))
                if split_last:
                    run_chunk(tables[nchunks - 1],
                              outs[nchunks - 1 + core_val], (core_val,))

    return agg


_agg1 = _make_agg(NCH1, False)
_agg2 = _make_agg(NCH2, True)


def _dis_of(c0_ref, c1_ref):
    cnt = c0_ref[:, :1] + c1_ref[:, :1]
    return lax.rsqrt(cnt + 1.0)


def _b1_body(*refs):
    x_ref, w_ref, c0_ref, c1_ref = refs[:4]
    outs = refs[4:]
    dis = _dis_of(c0_ref, c1_ref)
    h = lax.dot_general(x_ref[...], w_ref[...], (((0,), (0,)), ((), ())),
                        preferred_element_type=_F32)
    hp = h * dis
    for c, o in enumerate(outs):
        o[...] = hp[:, c * CW:(c + 1) * CW]


_b1_call = pl.pallas_call(
    _b1_body,
    grid=(GRID,),
    in_specs=[
        pl.BlockSpec((KP1, RBLK), lambda i: (0, i)),
        pl.BlockSpec((KP1, DHID), lambda i: (0, 0)),
        pl.BlockSpec((RBLK, CW), lambda i: (i, 0)),
        pl.BlockSpec((RBLK, CW), lambda i: (i, 0)),
    ],
    out_specs=[pl.BlockSpec((RBLK, CW), lambda i: (i, 0))
               for _ in range(NCH1)],
    out_shape=[jax.ShapeDtypeStruct((NPAD, CW), _F32) for _ in range(NCH1)],
)


def _b2_body(*refs):
    aggs = refs[:NCH1]
    hps = refs[NCH1:2 * NCH1]
    c0_ref, c1_ref, b1_ref, w_ref = refs[2 * NCH1:2 * NCH1 + 4]
    outs = refs[2 * NCH1 + 4:]
    dis = _dis_of(c0_ref, c1_ref)
    zs = []
    for c, (a, h) in enumerate(zip(aggs, hps)):
        pre = dis * (a[...] + h[...]) + b1_ref[0:1, c * CW:(c + 1) * CW]
        zs.append(jnp.maximum(pre, 0.0))
    z = jnp.concatenate(zs, axis=1)
    h2f = jnp.dot(z, w_ref[...], preferred_element_type=_F32)
    for c, o in enumerate(outs):
        o[...] = dis * h2f[:, c * CW:(c + 1) * CW]


_b2_call = pl.pallas_call(
    _b2_body,
    grid=(GRID,),
    in_specs=(
        [pl.BlockSpec((RBLK, CW), lambda i: (i, 0)) for _ in range(2 * NCH1)]
        + [pl.BlockSpec((RBLK, CW), lambda i: (i, 0)) for _ in range(2)]
        + [pl.BlockSpec((1, DHID), lambda i: (0, 0)),
           pl.BlockSpec((DHID, DP2), lambda i: (0, 0))]
    ),
    out_specs=[pl.BlockSpec((RBLK, CW), lambda i: (i, 0))
               for _ in range(NCH2)],
    out_shape=[jax.ShapeDtypeStruct((NPAD, CW), _F32) for _ in range(NCH2)],
)


def _b3_body(*refs):
    aggs = refs[:NCH2 + 1]
    hps = refs[NCH2 + 1:2 * NCH2 + 1]
    c0_ref, c1_ref, b2_ref = refs[2 * NCH2 + 1:2 * NCH2 + 4]
    o = refs[-1]
    dis = _dis_of(c0_ref, c1_ref)
    svals = [aggs[0][...], aggs[1][...], aggs[2][...] + aggs[3][...]]
    outs = []
    for c, (a, h) in enumerate(zip(svals, hps)):
        pre = dis * (a + h[...]) + b2_ref[0:1, c * CW:(c + 1) * CW]
        outs.append(jnp.maximum(pre, 0.0))
    o[...] = jnp.transpose(jnp.concatenate(outs, axis=1))


_b3_call = pl.pallas_call(
    _b3_body,
    grid=(GRID,),
    in_specs=(
        [pl.BlockSpec((RBLK, CW), lambda i: (i, 0)) for _ in range(2 * NCH2 + 1)]
        + [pl.BlockSpec((RBLK, CW), lambda i: (i, 0)) for _ in range(2)]
        + [pl.BlockSpec((1, DP2), lambda i: (0, 0))]
    ),
    out_specs=pl.BlockSpec((DP2, RBLK), lambda i: (0, i)),
    out_shape=jax.ShapeDtypeStruct((DP2, NPAD), _F32),
)


def kernel(x, edge_index, W1, b1, W2, b2):
    src = edge_index[0].astype(jnp.int32)
    dst = edge_index[1].astype(jnp.int32)
    pad = jnp.full((EPAD - E,), N, jnp.int32)
    srcp = jnp.concatenate([src, pad])
    dstp = jnp.concatenate([dst, pad])
    src16 = srcp.reshape(NS, 80, 128)
    dst16 = dstp.reshape(NS, 80, 128)
    dst32 = dstp.reshape(NC * NS, 40, 128)

    xtp = jnp.pad(x.astype(_F32).T, ((0, KP1 - x.shape[1]), (0, NPAD - N)))
    w1p = jnp.pad(W1.astype(_F32), ((0, KP1 - W1.shape[0]), (0, 0)))
    w2p = jnp.pad(W2.astype(_F32), ((0, 0), (0, DP2 - W2.shape[1])))
    b1r = b1.reshape(1, DHID).astype(_F32)
    b2r = jnp.pad(b2.astype(_F32), (0, DP2 - b2.shape[0])).reshape(1, DP2)

    c0, c1 = _deg_kernel(dst32)
    h1 = _b1_call(xtp, w1p, c0, c1)
    a1 = _agg1(src16, dst16, *h1)
    h2 = _b2_call(*a1, *h1, c0, c1, b1r, w2p)
    a2 = _agg2(src16, dst16, *h2)
    out = _b3_call(*a2, *h2, c0, c1, b2r)
    return out[:300, :N].T


# speedup vs baseline: 1.0364x; 1.0004x over previous
import functools

import jax
import jax.numpy as jnp
from jax import lax
from jax.experimental import pallas as pl
from jax.experimental.pallas import tpu as pltpu
from jax.experimental.pallas import tpu_sc as plsc

N = 10000
NPAD = 10240
E = 160000
EPAD = 163840
KP1 = 384
DHID = 512
DP2 = 384
CW = 128
NCH1 = DHID // CW
NCH2 = DP2 // CW
NC, NS = 2, 16
RPT = NPAD // NS
RBLK = 2048
GRID = NPAD // RBLK

_MESH = dict(core_axis_name="c", subcore_axis_name="s", num_cores=NC,
             num_subcores=NS)
_F32 = jnp.float32


@functools.partial(
    pl.kernel,
    out_type=[jax.ShapeDtypeStruct((NPAD, CW), _F32) for _ in range(NC)],
    mesh=plsc.VectorSubcoreMesh(**_MESH),
    scratch_types=[
        pltpu.VMEM((40, 128), jnp.int32),
        pltpu.VMEM((128, CW), _F32),
        pltpu.VMEM((16, CW), _F32),
        pltpu.VMEM_SHARED((NPAD, CW), _F32),
    ],
)
def _deg_kernel(dst_hbm, out0, out1, idx_v, ones_v, zero_v, acc):
    cid = lax.axis_index("c")
    sid = lax.axis_index("s")
    wid = sid * NC + cid

    def fill(i, _):
        for k16 in range(CW // 16):
            ones_v[i, pl.ds(k16 * 16, 16)] = jnp.ones((16,), _F32)
        return 0
    lax.fori_loop(0, 128, fill, 0)

    def fillz(i, _):
        for k16 in range(CW // 16):
            zero_v[i, pl.ds(k16 * 16, 16)] = jnp.zeros((16,), _F32)
        return 0
    lax.fori_loop(0, 16, fillz, 0)

    pltpu.sync_copy(dst_hbm.at[wid], idx_v)

    def zero(i, _):
        pltpu.sync_copy(zero_v, acc.at[pl.ds(sid * RPT + i * 16, 16)])
        return 0
    lax.fori_loop(0, RPT // 16, zero, 0)
    plsc.subcore_barrier()

    def scat(g, _):
        pltpu.sync_copy(ones_v, acc.at[idx_v.at[g]], add=True)
        return 0
    lax.fori_loop(0, 40, scat, 0)
    plsc.subcore_barrier()

    @pl.when(cid == 0)
    def _():
        pltpu.sync_copy(acc.at[pl.ds(sid * RPT, RPT)],
                        out0.at[pl.ds(sid * RPT, RPT)])

    @pl.when(cid == 1)
    def _():
        pltpu.sync_copy(acc.at[pl.ds(sid * RPT, RPT)],
                        out1.at[pl.ds(sid * RPT, RPT)])


def _make_agg(nchunks, split_last):
    nouts = nchunks + (1 if split_last else 0)

    @functools.partial(
        pl.kernel,
        out_type=[jax.ShapeDtypeStruct((NPAD, CW), _F32)
                  for _ in range(nouts)],
        mesh=plsc.VectorSubcoreMesh(**_MESH),
        scratch_types=[
            pltpu.VMEM((40, 128), jnp.int32),
            pltpu.VMEM((40, 128), jnp.int32),
            pltpu.VMEM((128, CW), _F32),
            pltpu.VMEM((128, CW), _F32),
            pltpu.VMEM((16, CW), _F32),
            pltpu.VMEM_SHARED((NPAD, CW), _F32),
            pltpu.SemaphoreType.DMA,
            pltpu.SemaphoreType.DMA,
            pltpu.SemaphoreType.DMA,
            pltpu.SemaphoreType.DMA,
        ],
    )
    def agg(*refs):
        src_hbm, dst_hbm = refs[0], refs[1]
        tables = refs[2:2 + nchunks]
        outs = refs[2 + nchunks:2 + nchunks + nouts]
        (src_v, dst_v, rows_a, rows_b, zero_v, acc,
         sem_a, sem_b, sem_sa, sem_sb) = refs[2 + nchunks + nouts:]
        cid = lax.axis_index("c")
        sid = lax.axis_index("s")

        def fz(i, _):
            for k16 in range(CW // 16):
                zero_v[i, pl.ds(k16 * 16, 16)] = jnp.zeros((16,), _F32)
            return 0
        lax.fori_loop(0, 16, fz, 0)

        def run_chunk(table, out, halves):
            def zero(i, _):
                pltpu.sync_copy(zero_v,
                                acc.at[pl.ds(sid * RPT + i * 16, 16)])
                return 0
            lax.fori_loop(0, RPT // 16, zero, 0)
            plsc.subcore_barrier()

            for half in halves:
                pltpu.sync_copy(src_hbm.at[sid, pl.ds(half * 40, 40)],
                                src_v)
                pltpu.sync_copy(dst_hbm.at[sid, pl.ds(half * 40, 40)],
                                dst_v)

                def g_start(buf, sem, g):
                    pltpu.async_copy(table.at[src_v.at[g]], buf, sem)

                def g_wait(buf, sem):
                    pltpu.make_async_copy(table.at[src_v.at[0]], buf,
                                          sem).wait()

                def s_start(buf, sem, g):
                    pltpu.async_copy(buf, acc.at[dst_v.at[g]], sem,
                                     add=True)

                def s_wait(buf, sem):
                    pltpu.make_async_copy(buf, acc.at[dst_v.at[0]],
                                          sem).wait()

                g_start(rows_a, sem_a, 0)
                g_start(rows_b, sem_b, 1)

                def body(gg, _):
                    g0 = 2 * gg
                    g_wait(rows_a, sem_a)
                    s_start(rows_a, sem_sa, g0)
                    g_wait(rows_b, sem_b)
                    s_start(rows_b, sem_sb, g0 + 1)
                    s_wait(rows_a, sem_sa)

                    @pl.when(gg < 19)
                    def _():
                        g_start(rows_a, sem_a, g0 + 2)
                    s_wait(rows_b, sem_sb)

                    @pl.when(gg < 19)
                    def _():
                        g_start(rows_b, sem_b, g0 + 3)
                    return 0
                lax.fori_loop(0, 20, body, 0)

            plsc.subcore_barrier()
            pltpu.sync_copy(acc.at[pl.ds(sid * RPT, RPT)],
                            out.at[pl.ds(sid * RPT, RPT)])
            plsc.subcore_barrier()

        half_chunks = nchunks // 2
        for core_val in range(NC):
            @pl.when(cid == core_val)
            def _():
                for j in range(half_chunks):
                    chunk = core_val * half_chunks + j
                    run_chunk(tables[chunk], outs[chunk], (0, 1))
                if split_last:
                    run_chunk(tables[nchunks - 1],
                              outs[nchunks - 1 + core_val], (core_val,))

    return agg


_agg1 = _make_agg(NCH1, False)
_agg2 = _make_agg(NCH2, True)


def _dis_of(c0_ref, c1_ref):
    cnt = c0_ref[:, :1] + c1_ref[:, :1]
    return lax.rsqrt(cnt + 1.0)


def _b1_body(*refs):
    x_ref, w_ref, c0_ref, c1_ref = refs[:4]
    outs = refs[4:]
    dis = _dis_of(c0_ref, c1_ref)
    h = lax.dot_general(x_ref[...], w_ref[...], (((0,), (0,)), ((), ())),
                        preferred_element_type=_F32)
    hp = h * dis
    for c, o in enumerate(outs):
        o[...] = hp[:, c * CW:(c + 1) * CW]


_b1_call = pl.pallas_call(
    _b1_body,
    grid=(GRID,),
    in_specs=[
        pl.BlockSpec((KP1, RBLK), lambda i: (0, i)),
        pl.BlockSpec((KP1, DHID), lambda i: (0, 0)),
        pl.BlockSpec((RBLK, CW), lambda i: (i, 0)),
        pl.BlockSpec((RBLK, CW), lambda i: (i, 0)),
    ],
    out_specs=[pl.BlockSpec((RBLK, CW), lambda i: (i, 0))
               for _ in range(NCH1)],
    out_shape=[jax.ShapeDtypeStruct((NPAD, CW), _F32) for _ in range(NCH1)],
)


def _b2_body(*refs):
    aggs = refs[:NCH1]
    hps = refs[NCH1:2 * NCH1]
    c0_ref, c1_ref, b1_ref, w_ref = refs[2 * NCH1:2 * NCH1 + 4]
    outs = refs[2 * NCH1 + 4:]
    dis = _dis_of(c0_ref, c1_ref)
    zs = []
    for c, (a, h) in enumerate(zip(aggs, hps)):
        pre = dis * (a[...] + h[...]) + b1_ref[0:1, c * CW:(c + 1) * CW]
        zs.append(jnp.maximum(pre, 0.0))
    z = jnp.concatenate(zs, axis=1)
    h2f = jnp.dot(z, w_ref[...], preferred_element_type=_F32)
    for c, o in enumerate(outs):
        o[...] = dis * h2f[:, c * CW:(c + 1) * CW]


_b2_call = pl.pallas_call(
    _b2_body,
    grid=(GRID,),
    in_specs=(
        [pl.BlockSpec((RBLK, CW), lambda i: (i, 0)) for _ in range(2 * NCH1)]
        + [pl.BlockSpec((RBLK, CW), lambda i: (i, 0)) for _ in range(2)]
        + [pl.BlockSpec((1, DHID), lambda i: (0, 0)),
           pl.BlockSpec((DHID, DP2), lambda i: (0, 0))]
    ),
    out_specs=[pl.BlockSpec((RBLK, CW), lambda i: (i, 0))
               for _ in range(NCH2)],
    out_shape=[jax.ShapeDtypeStruct((NPAD, CW), _F32) for _ in range(NCH2)],
)


def _b3_body(*refs):
    aggs = refs[:NCH2 + 1]
    hps = refs[NCH2 + 1:2 * NCH2 + 1]
    c0_ref, c1_ref, b2_ref = refs[2 * NCH2 + 1:2 * NCH2 + 4]
    o = refs[-1]
    dis = _dis_of(c0_ref, c1_ref)
    svals = [aggs[0][...], aggs[1][...], aggs[2][...] + aggs[3][...]]
    outs = []
    for c, (a, h) in enumerate(zip(svals, hps)):
        pre = dis * (a + h[...]) + b2_ref[0:1, c * CW:(c + 1) * CW]
        outs.append(jnp.maximum(pre, 0.0))
    o[...] = jnp.transpose(jnp.concatenate(outs, axis=1))


_b3_call = pl.pallas_call(
    _b3_body,
    grid=(GRID,),
    in_specs=(
        [pl.BlockSpec((RBLK, CW), lambda i: (i, 0)) for _ in range(2 * NCH2 + 1)]
        + [pl.BlockSpec((RBLK, CW), lambda i: (i, 0)) for _ in range(2)]
        + [pl.BlockSpec((1, DP2), lambda i: (0, 0))]
    ),
    out_specs=pl.BlockSpec((DP2, RBLK), lambda i: (0, i)),
    out_shape=jax.ShapeDtypeStruct((DP2, NPAD), _F32),
)


def kernel(x, edge_index, W1, b1, W2, b2):
    src = edge_index[0].astype(jnp.int32)
    dst = edge_index[1].astype(jnp.int32)
    pad = jnp.full((EPAD - E,), N, jnp.int32)
    srcp = jnp.concatenate([src, pad])
    dstp = jnp.concatenate([dst, pad])
    src16 = srcp.reshape(NS, 80, 128)
    dst16 = dstp.reshape(NS, 80, 128)
    dst32 = dstp.reshape(NC * NS, 40, 128)

    xtp = jnp.pad(x.astype(_F32).T, ((0, KP1 - x.shape[1]), (0, NPAD - N)))
    w1p = jnp.pad(W1.astype(_F32), ((0, KP1 - W1.shape[0]), (0, 0)))
    w2p = jnp.pad(W2.astype(_F32), ((0, 0), (0, DP2 - W2.shape[1])))
    b1r = b1.reshape(1, DHID).astype(_F32)
    b2r = jnp.pad(b2.astype(_F32), (0, DP2 - b2.shape[0])).reshape(1, DP2)

    c0, c1 = _deg_kernel(dst32)
    h1 = _b1_call(xtp, w1p, c0, c1)
    a1 = _agg1(src16, dst16, *h1)
    h2 = _b2_call(*a1, *h1, c0, c1, b1r, w2p)
    a2 = _agg2(src16, dst16, *h2)
    out = _b3_call(*a2, *h2, c0, c1, b2r)
    return out[:300, :N].T


# speedup vs baseline: 1.0472x; 1.0104x over previous
import functools

import jax
import jax.numpy as jnp
from jax import lax
from jax.experimental import pallas as pl
from jax.experimental.pallas import tpu as pltpu
from jax.experimental.pallas import tpu_sc as plsc

N = 10000
NPAD = 10240
E = 160000
EPAD = 163840
KP1 = 384
DHID = 512
DP2 = 384
CW = 128
NCH1 = DHID // CW
NCH2 = DP2 // CW
NC, NS = 2, 16
RPT = NPAD // NS
RBLK = 2048
GRID = NPAD // RBLK

_MESH = dict(core_axis_name="c", subcore_axis_name="s", num_cores=NC,
             num_subcores=NS)
_F32 = jnp.float32


@functools.partial(
    pl.kernel,
    out_type=[jax.ShapeDtypeStruct((NPAD, CW), _F32) for _ in range(NC)],
    mesh=plsc.VectorSubcoreMesh(**_MESH),
    scratch_types=[
        pltpu.VMEM((40, 128), jnp.int32),
        pltpu.VMEM((128, CW), _F32),
        pltpu.VMEM((16, CW), _F32),
        pltpu.VMEM_SHARED((NPAD, CW), _F32),
        pltpu.SemaphoreType.DMA,
    ],
)
def _deg_kernel(dst_hbm, out0, out1, idx_v, ones_v, zero_v, acc, sem_z):
    cid = lax.axis_index("c")
    sid = lax.axis_index("s")
    wid = sid * NC + cid

    def fill(i, _):
        for k16 in range(CW // 16):
            ones_v[i, pl.ds(k16 * 16, 16)] = jnp.ones((16,), _F32)
        return 0
    lax.fori_loop(0, 128, fill, 0)

    def fillz(i, _):
        for k16 in range(CW // 16):
            zero_v[i, pl.ds(k16 * 16, 16)] = jnp.zeros((16,), _F32)
        return 0
    lax.fori_loop(0, 16, fillz, 0)

    pltpu.sync_copy(dst_hbm.at[wid], idx_v)

    def zero(i, _):
        pltpu.async_copy(zero_v, acc.at[pl.ds(sid * RPT + i * 16, 16)],
                         sem_z)
        return 0
    lax.fori_loop(0, RPT // 16, zero, 0)

    def zdrain(i, _):
        pltpu.make_async_copy(zero_v, acc.at[pl.ds(sid * RPT, 16)],
                              sem_z).wait()
        return 0
    lax.fori_loop(0, RPT // 16, zdrain, 0)
    plsc.subcore_barrier()

    def scat(g, _):
        pltpu.sync_copy(ones_v, acc.at[idx_v.at[g]], add=True)
        return 0
    lax.fori_loop(0, 40, scat, 0)
    plsc.subcore_barrier()

    @pl.when(cid == 0)
    def _():
        pltpu.sync_copy(acc.at[pl.ds(sid * RPT, RPT)],
                        out0.at[pl.ds(sid * RPT, RPT)])

    @pl.when(cid == 1)
    def _():
        pltpu.sync_copy(acc.at[pl.ds(sid * RPT, RPT)],
                        out1.at[pl.ds(sid * RPT, RPT)])


def _make_agg(nchunks, split_last):
    nouts = nchunks + (1 if split_last else 0)

    @functools.partial(
        pl.kernel,
        out_type=[jax.ShapeDtypeStruct((NPAD, CW), _F32)
                  for _ in range(nouts)],
        mesh=plsc.VectorSubcoreMesh(**_MESH),
        scratch_types=[
            pltpu.VMEM((40, 128), jnp.int32),
            pltpu.VMEM((40, 128), jnp.int32),
            pltpu.VMEM((128, CW), _F32),
            pltpu.VMEM((128, CW), _F32),
            pltpu.VMEM((16, CW), _F32),
            pltpu.VMEM_SHARED((NPAD, CW), _F32),
            pltpu.SemaphoreType.DMA,
            pltpu.SemaphoreType.DMA,
            pltpu.SemaphoreType.DMA,
            pltpu.SemaphoreType.DMA,
            pltpu.SemaphoreType.DMA,
        ],
    )
    def agg(*refs):
        src_hbm, dst_hbm = refs[0], refs[1]
        tables = refs[2:2 + nchunks]
        outs = refs[2 + nchunks:2 + nchunks + nouts]
        (src_v, dst_v, rows_a, rows_b, zero_v, acc,
         sem_a, sem_b, sem_sa, sem_sb, sem_z) = refs[2 + nchunks + nouts:]
        cid = lax.axis_index("c")
        sid = lax.axis_index("s")

        def fz(i, _):
            for k16 in range(CW // 16):
                zero_v[i, pl.ds(k16 * 16, 16)] = jnp.zeros((16,), _F32)
            return 0
        lax.fori_loop(0, 16, fz, 0)

        def run_chunk(table, out, halves):
            def zero(i, _):
                pltpu.async_copy(zero_v,
                                 acc.at[pl.ds(sid * RPT + i * 16, 16)],
                                 sem_z)
                return 0
            lax.fori_loop(0, RPT // 16, zero, 0)

            def zdrain(i, _):
                pltpu.make_async_copy(
                    zero_v, acc.at[pl.ds(sid * RPT, 16)], sem_z).wait()
                return 0
            lax.fori_loop(0, RPT // 16, zdrain, 0)
            plsc.subcore_barrier()

            for half in halves:
                pltpu.sync_copy(src_hbm.at[sid, pl.ds(half * 40, 40)],
                                src_v)
                pltpu.sync_copy(dst_hbm.at[sid, pl.ds(half * 40, 40)],
                                dst_v)

                def g_start(buf, sem, g):
                    pltpu.async_copy(table.at[src_v.at[g]], buf, sem)

                def g_wait(buf, sem):
                    pltpu.make_async_copy(table.at[src_v.at[0]], buf,
                                          sem).wait()

                def s_start(buf, sem, g):
                    pltpu.async_copy(buf, acc.at[dst_v.at[g]], sem,
                                     add=True)

                def s_wait(buf, sem):
                    pltpu.make_async_copy(buf, acc.at[dst_v.at[0]],
                                          sem).wait()

                g_start(rows_a, sem_a, 0)
                g_start(rows_b, sem_b, 1)

                def body(gg, _):
                    g0 = 2 * gg
                    g_wait(rows_a, sem_a)
                    s_start(rows_a, sem_sa, g0)
                    g_wait(rows_b, sem_b)
                    s_start(rows_b, sem_sb, g0 + 1)
                    s_wait(rows_a, sem_sa)

                    @pl.when(gg < 19)
                    def _():
                        g_start(rows_a, sem_a, g0 + 2)
                    s_wait(rows_b, sem_sb)

                    @pl.when(gg < 19)
                    def _():
                        g_start(rows_b, sem_b, g0 + 3)
                    return 0
                lax.fori_loop(0, 20, body, 0)

            plsc.subcore_barrier()
            pltpu.sync_copy(acc.at[pl.ds(sid * RPT, RPT)],
                            out.at[pl.ds(sid * RPT, RPT)])
            plsc.subcore_barrier()

        half_chunks = nchunks // 2
        for core_val in range(NC):
            @pl.when(cid == core_val)
            def _():
                for j in range(half_chunks):
                    chunk = core_val * half_chunks + j
                    run_chunk(tables[chunk], outs[chunk], (0, 1))
                if split_last:
                    run_chunk(tables[nchunks - 1],
                              outs[nchunks - 1 + core_val], (core_val,))

    return agg


_agg1 = _make_agg(NCH1, False)
_agg2 = _make_agg(NCH2, True)


def _dis_of(c0_ref, c1_ref):
    cnt = c0_ref[:, :1] + c1_ref[:, :1]
    return lax.rsqrt(cnt + 1.0)


def _b1_body(*refs):
    x_ref, w_ref, c0_ref, c1_ref = refs[:4]
    outs = refs[4:]
    dis = _dis_of(c0_ref, c1_ref)
    h = lax.dot_general(x_ref[...], w_ref[...], (((0,), (0,)), ((), ())),
                        preferred_element_type=_F32)
    hp = h * dis
    for c, o in enumerate(outs):
        o[...] = hp[:, c * CW:(c + 1) * CW]


_b1_call = pl.pallas_call(
    _b1_body,
    grid=(GRID,),
    in_specs=[
        pl.BlockSpec((KP1, RBLK), lambda i: (0, i)),
        pl.BlockSpec((KP1, DHID), lambda i: (0, 0)),
        pl.BlockSpec((RBLK, CW), lambda i: (i, 0)),
        pl.BlockSpec((RBLK, CW), lambda i: (i, 0)),
    ],
    out_specs=[pl.BlockSpec((RBLK, CW), lambda i: (i, 0))
               for _ in range(NCH1)],
    out_shape=[jax.ShapeDtypeStruct((NPAD, CW), _F32) for _ in range(NCH1)],
)


def _b2_body(*refs):
    aggs = refs[:NCH1]
    hps = refs[NCH1:2 * NCH1]
    c0_ref, c1_ref, b1_ref, w_ref = refs[2 * NCH1:2 * NCH1 + 4]
    outs = refs[2 * NCH1 + 4:]
    dis = _dis_of(c0_ref, c1_ref)
    zs = []
    for c, (a, h) in enumerate(zip(aggs, hps)):
        pre = dis * (a[...] + h[...]) + b1_ref[0:1, c * CW:(c + 1) * CW]
        zs.append(jnp.maximum(pre, 0.0))
    z = jnp.concatenate(zs, axis=1)
    h2f = jnp.dot(z, w_ref[...], preferred_element_type=_F32)
    for c, o in enumerate(outs):
        o[...] = dis * h2f[:, c * CW:(c + 1) * CW]


_b2_call = pl.pallas_call(
    _b2_body,
    grid=(GRID,),
    in_specs=(
        [pl.BlockSpec((RBLK, CW), lambda i: (i, 0)) for _ in range(2 * NCH1)]
        + [pl.BlockSpec((RBLK, CW), lambda i: (i, 0)) for _ in range(2)]
        + [pl.BlockSpec((1, DHID), lambda i: (0, 0)),
           pl.BlockSpec((DHID, DP2), lambda i: (0, 0))]
    ),
    out_specs=[pl.BlockSpec((RBLK, CW), lambda i: (i, 0))
               for _ in range(NCH2)],
    out_shape=[jax.ShapeDtypeStruct((NPAD, CW), _F32) for _ in range(NCH2)],
)


def _b3_body(*refs):
    aggs = refs[:NCH2 + 1]
    hps = refs[NCH2 + 1:2 * NCH2 + 1]
    c0_ref, c1_ref, b2_ref = refs[2 * NCH2 + 1:2 * NCH2 + 4]
    o = refs[-1]
    dis = _dis_of(c0_ref, c1_ref)
    svals = [aggs[0][...], aggs[1][...], aggs[2][...] + aggs[3][...]]
    outs = []
    for c, (a, h) in enumerate(zip(svals, hps)):
        pre = dis * (a + h[...]) + b2_ref[0:1, c * CW:(c + 1) * CW]
        outs.append(jnp.maximum(pre, 0.0))
    o[...] = jnp.transpose(jnp.concatenate(outs, axis=1))


_b3_call = pl.pallas_call(
    _b3_body,
    grid=(GRID,),
    in_specs=(
        [pl.BlockSpec((RBLK, CW), lambda i: (i, 0)) for _ in range(2 * NCH2 + 1)]
        + [pl.BlockSpec((RBLK, CW), lambda i: (i, 0)) for _ in range(2)]
        + [pl.BlockSpec((1, DP2), lambda i: (0, 0))]
    ),
    out_specs=pl.BlockSpec((DP2, RBLK), lambda i: (0, i)),
    out_shape=jax.ShapeDtypeStruct((DP2, NPAD), _F32),
)


def kernel(x, edge_index, W1, b1, W2, b2):
    src = edge_index[0].astype(jnp.int32)
    dst = edge_index[1].astype(jnp.int32)
    pad = jnp.full((EPAD - E,), N, jnp.int32)
    srcp = jnp.concatenate([src, pad])
    dstp = jnp.concatenate([dst, pad])
    src16 = srcp.reshape(NS, 80, 128)
    dst16 = dstp.reshape(NS, 80, 128)
    dst32 = dstp.reshape(NC * NS, 40, 128)

    xtp = jnp.pad(x.astype(_F32).T, ((0, KP1 - x.shape[1]), (0, NPAD - N)))
    w1p = jnp.pad(W1.astype(_F32), ((0, KP1 - W1.shape[0]), (0, 0)))
    w2p = jnp.pad(W2.astype(_F32), ((0, 0), (0, DP2 - W2.shape[1])))
    b1r = b1.reshape(1, DHID).astype(_F32)
    b2r = jnp.pad(b2.astype(_F32), (0, DP2 - b2.shape[0])).reshape(1, DP2)

    c0, c1 = _deg_kernel(dst32)
    h1 = _b1_call(xtp, w1p, c0, c1)
    a1 = _agg1(src16, dst16, *h1)
    h2 = _b2_call(*a1, *h1, c0, c1, b1r, w2p)
    a2 = _agg2(src16, dst16, *h2)
    out = _b3_call(*a2, *h2, c0, c1, b2r)
    return out[:300, :N].T
